# baseline (speedup 1.0000x reference)
import jax
import jax.numpy as jnp
from jax.experimental import pallas as pl
from jax.experimental.pallas import tpu as pltpu

_EPS = 1e-5
_CB = 24
_LANES = 128


def _ln(x, gamma, beta):
    mean = jnp.mean(x, axis=-1, keepdims=True)
    var = jnp.mean((x - mean) ** 2, axis=-1, keepdims=True)
    return (x - mean) * jax.lax.rsqrt(var + _EPS) * gamma + beta


def _scale_body(tv_ref, tt_ref, vt_ref, tg_ref, tb_ref, vg_ref, vb_ref,
                scale_ref):
    B = tv_ref.shape[0]
    V = tt_ref.shape[0]
    idx = tv_ref[:]
    iota = jax.lax.broadcasted_iota(jnp.int32, (B, V), 1)
    oh_t = (iota == idx[:, 0:1]).astype(jnp.float32)
    oh_v = (iota == idx[:, 1:2]).astype(jnp.float32)
    temb = jnp.dot(oh_t, tt_ref[:], preferred_element_type=jnp.float32,
                   precision=jax.lax.Precision.HIGHEST)
    vemb = jnp.dot(oh_v, vt_ref[:], preferred_element_type=jnp.float32,
                   precision=jax.lax.Precision.HIGHEST)
    tln = _ln(temb, tg_ref[:], tb_ref[:])
    vln = _ln(vemb, vg_ref[:], vb_ref[:])
    scale_ref[:] = jnp.broadcast_to((tln * vln)[:, :, None],
                                    scale_ref.shape)


def _mul_body(ft_ref, scale_ref, out_ref):
    HW = ft_ref.shape[2]
    for v in range(HW // _LANES):
        sl = pl.ds(v * _LANES, _LANES)
        out_ref[:, :, sl] = ft_ref[:, :, sl] * scale_ref[:]


def kernel(ft, taskvar, task_table, var_table, task_gamma, task_beta,
           var_gamma, var_beta):
    B, C, H, W = ft.shape
    HW = H * W

    scale = pl.pallas_call(
        _scale_body,
        out_shape=jax.ShapeDtypeStruct((B, C, _LANES), jnp.float32),
    )(taskvar, task_table, var_table,
      task_gamma.reshape(1, C), task_beta.reshape(1, C),
      var_gamma.reshape(1, C), var_beta.reshape(1, C))

    ft3 = ft.reshape(B, C, HW)
    out3 = pl.pallas_call(
        _mul_body,
        grid=(C // _CB,),
        in_specs=[
            pl.BlockSpec((B, _CB, HW), lambda c: (0, c, 0)),
            pl.BlockSpec((B, _CB, _LANES), lambda c: (0, c, 0)),
        ],
        out_specs=pl.BlockSpec((B, _CB, HW), lambda c: (0, c, 0)),
        out_shape=jax.ShapeDtypeStruct((B, C, HW), ft.dtype),
    )(ft3, scale)
    return out3.reshape(B, C, H, W)
